# baseline (speedup 1.0000x reference)
import functools

import numpy as np
import jax
import jax.numpy as jnp
from jax import lax
from jax.experimental import pallas as pl
from jax.experimental.pallas import tpu as pltpu
from jax.experimental.pallas import tpu_sc as plsc

NQ = 8192
NG = 32
NSLOT = 8
NCAND = 27
C = 256
H1 = 512
QB = 256
STEP = np.float32(1.0 / 31.0)
R2 = np.float32(0.033 * 0.033)
_INV9 = np.float32(1.0 / 9.0)
_INV3 = np.float32(1.0 / 3.0)
_SQRT1_2 = np.float32(0.7071067811865476)


def _bf(x):
    return x.astype(jnp.bfloat16).astype(jnp.float32)


def _sum3_rn(p0, p1, p2):
    s1 = p0 + p1
    bp = s1 - p0
    ap = s1 - bp
    e1 = (p0 - ap) + (p1 - bp)
    s2 = s1 + p2
    bp2 = s2 - s1
    ap2 = s2 - bp2
    e2 = (s1 - ap2) + (p2 - bp2)
    return s2 + (e1 + e2)


WIN = 11
WOFF = 5
NYZ = WIN * WIN


def _search_kernel(qT_ref, nidx_ref, smask_ref, scnt_ref):
    nql = qT_ref.shape[1]
    qx = qT_ref[0:1, :]
    qy = qT_ref[1:2, :]
    qz = qT_ref[2:3, :]
    qbx = _bf(qx)
    qby = _bf(qy)
    qbz = _bf(qz)
    q2 = (qx * qx + qz * qz) + qy * qy
    bx = jnp.floor(qx * 31.0 + 0.5).astype(jnp.int32)
    by = jnp.floor(qy * 31.0 + 0.5).astype(jnp.int32)
    bz = jnp.floor(qz * 31.0 + 0.5).astype(jnp.int32)

    def axis_rows(b, qf, qbf):
        idx, ybf, prod, sq, inb = [], [], [], [], []
        for o in range(-WOFF, WOFF + 1):
            ia = b + o
            ya = ia.astype(jnp.float32) * STEP
            yb = _bf(ya)
            idx.append(ia)
            ybf.append(yb)
            prod.append(qbf * yb)
            sq.append(ya * ya)
            inb.append((ia >= 0) & (ia <= 31))
        return idx, ybf, prod, sq, inb

    ix_r, _, px_r, sqx_r, inbx_r = axis_rows(bx, qx, qbx)
    iy_r, _, py_r, sqy_r, inby_r = axis_rows(by, qy, qby)
    iz_r, _, pz_r, sqz_r, inbz_r = axis_rows(bz, qz, qbz)

    d2n_list, valid_list, flat_list = [], [], []
    for dx in (-1, 0, 1):
        for dy in (-1, 0, 1):
            for dz in (-1, 0, 1):
                ox, oy, oz = dx + WOFF, dy + WOFF, dz + WOFF
                yxv = ix_r[ox].astype(jnp.float32) * STEP
                yyv = iy_r[oy].astype(jnp.float32) * STEP
                yzv = iz_r[oz].astype(jnp.float32) * STEP
                ddx = qx - yxv
                ddy = qy - yyv
                ddz = qz - yzv
                d2e = (ddx * ddx + ddz * ddz) + ddy * ddy
                inb = inbx_r[ox] & inby_r[oy] & inbz_r[oz]
                valid_list.append(inb & (d2e <= R2))
                y2v = (sqx_r[ox] + sqz_r[oz]) + sqy_r[oy]
                mm = _sum3_rn(px_r[ox], py_r[oy], pz_r[oz])
                d2n_list.append((q2 + y2v) - 2.0 * mm)
                flat_list.append((ix_r[ox] * 1024 + iy_r[oy] * 32) + iz_r[oz])
    vf = jnp.concatenate([v.astype(jnp.float32) for v in valid_list], axis=0)
    d2n_c = jnp.concatenate(d2n_list, axis=0)
    flat_c = jnp.concatenate(flat_list, axis=0)
    r = lax.broadcasted_iota(jnp.int32, (NCAND, NCAND), 0)
    cc = lax.broadcasted_iota(jnp.int32, (NCAND, NCAND), 1)
    L = (r > cc).astype(jnp.float32)
    P = jnp.dot(L, vf, preferred_element_type=jnp.float32)
    validb = vf > 0.5
    flat_m = jnp.where(validb, flat_c, 0)
    cnt = jnp.sum(vf, axis=0, keepdims=True)
    slot_flat, slot_d2n = [], []
    for s in range(NSLOT):
        msk = validb & (P == np.float32(s))
        slot_flat.append(jnp.sum(jnp.where(msk, flat_m, 0), axis=0,
                                 keepdims=True))
        slot_d2n.append(jnp.sum(jnp.where(msk, d2n_c, 0.0), axis=0,
                                keepdims=True))
        nidx_ref[s:s + 1, :] = slot_flat[s]

    def stack_yz(rows_y, rows_z, combine):
        ys = jnp.concatenate(
            [jnp.broadcast_to(ry, (WIN, nql)) for ry in rows_y], axis=0)
        zs = jnp.concatenate([jnp.concatenate(rows_z, axis=0)] * WIN, axis=0)
        return combine(ys, zs)

    PY = stack_yz(py_r, pz_r, lambda a, b: (a, b))
    PYa, PZa = PY
    SQZY = stack_yz(sqy_r, sqz_r, lambda a, b: (a, b))
    SQYa, SQZa = SQZY
    INByz = stack_yz([v.astype(jnp.float32) for v in inby_r],
                     [v.astype(jnp.float32) for v in inbz_r],
                     lambda a, b: (a > 0.5) & (b > 0.5))
    FLATyz = stack_yz([v * 32 for v in iy_r], iz_r, lambda a, b: a + b)

    ranks = [jnp.zeros((1, nql), jnp.float32) for _ in range(NSLOT)]
    for ox in range(WIN):
        y2w = (sqx_r[ox] + SQZa) + SQYa
        mmw = _sum3_rn(jnp.broadcast_to(px_r[ox], (NYZ, nql)), PYa, PZa)
        d2nw = (q2 + y2w) - 2.0 * mmw
        inbw = INByz & inbx_r[ox]
        flatw = ix_r[ox] * 1024 + FLATyz
        for s in range(NSLOT):
            tj = slot_d2n[s]
            fj = slot_flat[s]
            beats = (d2nw < tj) | ((d2nw == tj) & (flatw < fj))
            ranks[s] = ranks[s] + jnp.sum(
                jnp.where(inbw & beats, 1.0, 0.0), axis=0, keepdims=True)

    scnt = jnp.zeros((1, nql), jnp.float32)
    for s in range(NSLOT):
        sm = ((cnt > np.float32(s)) & (ranks[s] < 15.5)).astype(jnp.float32)
        smask_ref[s:s + 1, :] = sm
        scnt = scnt + sm
    scnt_ref[0:1, :] = scnt


def _search(qT):
    nchunk = 4
    nql = NQ // nchunk
    return pl.pallas_call(
        _search_kernel,
        grid=(nchunk,),
        in_specs=[pl.BlockSpec((3, nql), lambda i: (0, i))],
        out_specs=[
            pl.BlockSpec((NSLOT, nql), lambda i: (0, i)),
            pl.BlockSpec((NSLOT, nql), lambda i: (0, i)),
            pl.BlockSpec((1, nql), lambda i: (0, i)),
        ],
        out_shape=[
            jax.ShapeDtypeStruct((NSLOT, NQ), jnp.int32),
            jax.ShapeDtypeStruct((NSLOT, NQ), jnp.float32),
            jax.ShapeDtypeStruct((1, NQ), jnp.float32),
        ],
    )(qT)


def _sc_gather(table, idx):
    B = idx.shape[0]
    NW = 32
    BPW = B // NW
    CH = 128
    mesh = plsc.VectorSubcoreMesh(core_axis_name="c", subcore_axis_name="s")

    NCH = BPW // CH

    @functools.partial(
        pl.kernel, mesh=mesh,
        out_type=jax.ShapeDtypeStruct((B, C), jnp.float32),
        scratch_types=[
            pltpu.VMEM((BPW,), jnp.int32),
            pltpu.VMEM((CH, C), jnp.float32),
            pltpu.VMEM((CH, C), jnp.float32),
            pltpu.SemaphoreType.DMA,
            pltpu.SemaphoreType.DMA,
        ],
    )
    def k(table_hbm, idx_hbm, out_hbm, idx_v, rows0, rows1, g0, g1):
        wid = lax.axis_index("s") * 2 + lax.axis_index("c")
        base = wid * BPW
        pltpu.sync_copy(idx_hbm.at[pl.ds(base, BPW)], idx_v)
        bufs = ((rows0, g0), (rows1, g1))

        def gather_start(t, rows, sem):
            pltpu.make_async_copy(
                table_hbm.at[idx_v.at[pl.ds(t * CH, CH)]], rows, sem).start()

        for b in range(2):
            gather_start(b, *bufs[b])

        @pl.loop(0, NCH, step=2)
        def _(t):
            for b in range(2):
                rows, sem = bufs[b]
                tt = t + b
                pltpu.make_async_copy(
                    table_hbm.at[idx_v.at[pl.ds(tt * CH, CH)]], rows,
                    sem).wait()
                pltpu.sync_copy(rows, out_hbm.at[pl.ds(base + tt * CH, CH)])

                @pl.when(tt + 2 < NCH)
                def _():
                    gather_start(tt + 2, rows, sem)

    return k(table, idx)


def _gelu(x):
    return x * 0.5 * (1.0 + lax.erf(x * _SQRT1_2))


def _mlp_kernel(q_ref, nidxT_ref, smaskT_ref, scnt_ref, fN_ref, W1_ref, b1_ref,
                W2_ref, b2_ref, W3_ref, b3_ref, Wp_ref, bp_ref, out_ref):
    q = q_ref[...]
    scnt = scnt_ref[...]
    W1 = W1_ref[...]
    W2 = W2_ref[...]
    W3 = W3_ref[...]
    xp = (q[:, 0:1] * W1[3:4, :] + q[:, 1:2] * W1[4:5, :]
          + q[:, 2:3] * W1[5:6, :] + b1_ref[...])
    acc = jnp.zeros((QB, C), jnp.float32)
    for s in range(NSLOT):
        fl = nidxT_ref[:, s:s + 1]
        ixf = (fl >> 10).astype(jnp.float32)
        iyf = ((fl >> 5) & 31).astype(jnp.float32)
        izf = (fl & 31).astype(jnp.float32)
        h = (xp + (ixf * STEP) * W1[0:1, :] + (iyf * STEP) * W1[1:2, :]
             + (izf * STEP) * W1[2:3, :])
        h = _gelu(h)
        h = _gelu(jnp.dot(h, W2, preferred_element_type=jnp.float32)
                  + b2_ref[...])
        kern = (jnp.dot(h, W3, preferred_element_type=jnp.float32)
                + b3_ref[...])
        m = smaskT_ref[:, s:s + 1]
        acc = acc + kern * fN_ref[s] * m
    agg = acc / jnp.maximum(scnt, 1.0)
    out_ref[...] = (jnp.dot(agg, Wp_ref[...], preferred_element_type=jnp.float32)
                    + bp_ref[...])


def _mlp(q, nidxT, smaskT, scntT, fN, W1, b1, W2, b2, W3, b3, Wp, bp):
    grid = (NQ // QB,)
    return pl.pallas_call(
        _mlp_kernel,
        grid=grid,
        in_specs=[
            pl.BlockSpec((QB, 3), lambda i: (i, 0)),
            pl.BlockSpec((QB, NSLOT), lambda i: (i, 0)),
            pl.BlockSpec((QB, NSLOT), lambda i: (i, 0)),
            pl.BlockSpec((QB, 1), lambda i: (i, 0)),
            pl.BlockSpec((NSLOT, QB, C), lambda i: (0, i, 0)),
            pl.BlockSpec((6, H1), lambda i: (0, 0)),
            pl.BlockSpec((1, H1), lambda i: (0, 0)),
            pl.BlockSpec((H1, C), lambda i: (0, 0)),
            pl.BlockSpec((1, C), lambda i: (0, 0)),
            pl.BlockSpec((C, C), lambda i: (0, 0)),
            pl.BlockSpec((1, C), lambda i: (0, 0)),
            pl.BlockSpec((C, 4), lambda i: (0, 0)),
            pl.BlockSpec((1, 4), lambda i: (0, 0)),
        ],
        out_specs=pl.BlockSpec((QB, 4), lambda i: (i, 0)),
        out_shape=jax.ShapeDtypeStruct((NQ, 4), jnp.float32),
    )(q, nidxT, smaskT, scntT, fN, W1, b1, W2, b2, W3, b3, Wp, bp)


def kernel(latent_embed, latent_queries, output_queries,
           W1, b1, W2, b2, W3, b3, Wp, bp):
    del latent_queries
    q = output_queries[0]
    f_y = latent_embed.reshape(-1, C)
    nidx, smask, scnt = _search(q.T)
    fN = _sc_gather(f_y, nidx.reshape(-1))
    out = _mlp(q, nidx.T, smask.T, scnt.reshape(NQ, 1),
               fN.reshape(NSLOT, NQ, C), W1, b1.reshape(1, H1), W2,
               b2.reshape(1, C), W3, b3.reshape(1, C), Wp, bp.reshape(1, 4))
    return out[None]


# speedup vs baseline: 1.0008x; 1.0008x over previous
import functools

import numpy as np
import jax
import jax.numpy as jnp
from jax import lax
from jax.experimental import pallas as pl
from jax.experimental.pallas import tpu as pltpu
from jax.experimental.pallas import tpu_sc as plsc

NQ = 8192
NG = 32
NSLOT = 8
NCAND = 27
C = 256
H1 = 512
QB = 256
STEP = np.float32(1.0 / 31.0)
R2 = np.float32(0.033 * 0.033)
_INV9 = np.float32(1.0 / 9.0)
_INV3 = np.float32(1.0 / 3.0)
_SQRT1_2 = np.float32(0.7071067811865476)


def _bf(x):
    return x.astype(jnp.bfloat16).astype(jnp.float32)


def _sum3_rn(p0, p1, p2):
    s1 = p0 + p1
    bp = s1 - p0
    ap = s1 - bp
    e1 = (p0 - ap) + (p1 - bp)
    s2 = s1 + p2
    bp2 = s2 - s1
    ap2 = s2 - bp2
    e2 = (s1 - ap2) + (p2 - bp2)
    return s2 + (e1 + e2)


WIN = 11
WOFF = 5
NYZ = WIN * WIN


def _search_kernel(qT_ref, nidx_ref, smask_ref, scnt_ref):
    nql = qT_ref.shape[1]
    qx = qT_ref[0:1, :]
    qy = qT_ref[1:2, :]
    qz = qT_ref[2:3, :]
    qbx = _bf(qx)
    qby = _bf(qy)
    qbz = _bf(qz)
    q2 = (qx * qx + qz * qz) + qy * qy
    bx = jnp.floor(qx * 31.0 + 0.5).astype(jnp.int32)
    by = jnp.floor(qy * 31.0 + 0.5).astype(jnp.int32)
    bz = jnp.floor(qz * 31.0 + 0.5).astype(jnp.int32)

    def axis_rows(b, qf, qbf):
        idx, ybf, prod, sq, inb = [], [], [], [], []
        for o in range(-WOFF, WOFF + 1):
            ia = b + o
            ya = ia.astype(jnp.float32) * STEP
            yb = _bf(ya)
            idx.append(ia)
            ybf.append(yb)
            prod.append(qbf * yb)
            sq.append(ya * ya)
            inb.append((ia >= 0) & (ia <= 31))
        return idx, ybf, prod, sq, inb

    ix_r, _, px_r, sqx_r, inbx_r = axis_rows(bx, qx, qbx)
    iy_r, _, py_r, sqy_r, inby_r = axis_rows(by, qy, qby)
    iz_r, _, pz_r, sqz_r, inbz_r = axis_rows(bz, qz, qbz)

    d2n_list, valid_list, flat_list = [], [], []
    for dx in (-1, 0, 1):
        for dy in (-1, 0, 1):
            for dz in (-1, 0, 1):
                ox, oy, oz = dx + WOFF, dy + WOFF, dz + WOFF
                yxv = ix_r[ox].astype(jnp.float32) * STEP
                yyv = iy_r[oy].astype(jnp.float32) * STEP
                yzv = iz_r[oz].astype(jnp.float32) * STEP
                ddx = qx - yxv
                ddy = qy - yyv
                ddz = qz - yzv
                d2e = (ddx * ddx + ddz * ddz) + ddy * ddy
                inb = inbx_r[ox] & inby_r[oy] & inbz_r[oz]
                valid_list.append(inb & (d2e <= R2))
                y2v = (sqx_r[ox] + sqz_r[oz]) + sqy_r[oy]
                mm = _sum3_rn(px_r[ox], py_r[oy], pz_r[oz])
                d2n_list.append((q2 + y2v) - 2.0 * mm)
                flat_list.append((ix_r[ox] * 1024 + iy_r[oy] * 32) + iz_r[oz])
    vf = jnp.concatenate([v.astype(jnp.float32) for v in valid_list], axis=0)
    d2n_c = jnp.concatenate(d2n_list, axis=0)
    flat_c = jnp.concatenate(flat_list, axis=0)
    r = lax.broadcasted_iota(jnp.int32, (NCAND, NCAND), 0)
    cc = lax.broadcasted_iota(jnp.int32, (NCAND, NCAND), 1)
    L = (r > cc).astype(jnp.float32)
    P = jnp.dot(L, vf, preferred_element_type=jnp.float32)
    validb = vf > 0.5
    flat_m = jnp.where(validb, flat_c, 0)
    cnt = jnp.sum(vf, axis=0, keepdims=True)
    slot_flat, slot_d2n = [], []
    for s in range(NSLOT):
        msk = validb & (P == np.float32(s))
        slot_flat.append(jnp.sum(jnp.where(msk, flat_m, 0), axis=0,
                                 keepdims=True))
        slot_d2n.append(jnp.sum(jnp.where(msk, d2n_c, 0.0), axis=0,
                                keepdims=True))
        nidx_ref[s:s + 1, :] = slot_flat[s]

    def stack_yz(rows_y, rows_z, combine):
        ys = jnp.concatenate(
            [jnp.broadcast_to(ry, (WIN, nql)) for ry in rows_y], axis=0)
        zs = jnp.concatenate([jnp.concatenate(rows_z, axis=0)] * WIN, axis=0)
        return combine(ys, zs)

    PY = stack_yz(py_r, pz_r, lambda a, b: (a, b))
    PYa, PZa = PY
    SQZY = stack_yz(sqy_r, sqz_r, lambda a, b: (a, b))
    SQYa, SQZa = SQZY
    INByz = stack_yz([v.astype(jnp.float32) for v in inby_r],
                     [v.astype(jnp.float32) for v in inbz_r],
                     lambda a, b: (a > 0.5) & (b > 0.5))
    FLATyz = stack_yz([v * 32 for v in iy_r], iz_r, lambda a, b: a + b)

    ranks = [jnp.zeros((1, nql), jnp.float32) for _ in range(NSLOT)]
    for ox in range(WIN):
        y2w = (sqx_r[ox] + SQZa) + SQYa
        mmw = _sum3_rn(jnp.broadcast_to(px_r[ox], (NYZ, nql)), PYa, PZa)
        d2nw = (q2 + y2w) - 2.0 * mmw
        inbw = INByz & inbx_r[ox]
        flatw = ix_r[ox] * 1024 + FLATyz
        for s in range(NSLOT):
            tj = slot_d2n[s]
            fj = slot_flat[s]
            beats = (d2nw < tj) | ((d2nw == tj) & (flatw < fj))
            ranks[s] = ranks[s] + jnp.sum(
                jnp.where(inbw & beats, 1.0, 0.0), axis=0, keepdims=True)

    scnt = jnp.zeros((1, nql), jnp.float32)
    for s in range(NSLOT):
        sm = ((cnt > np.float32(s)) & (ranks[s] < 15.5)).astype(jnp.float32)
        smask_ref[s:s + 1, :] = sm
        scnt = scnt + sm
    scnt_ref[0:1, :] = scnt


def _search(qT):
    nchunk = 4
    nql = NQ // nchunk
    return pl.pallas_call(
        _search_kernel,
        grid=(nchunk,),
        in_specs=[pl.BlockSpec((3, nql), lambda i: (0, i))],
        out_specs=[
            pl.BlockSpec((NSLOT, nql), lambda i: (0, i)),
            pl.BlockSpec((NSLOT, nql), lambda i: (0, i)),
            pl.BlockSpec((1, nql), lambda i: (0, i)),
        ],
        out_shape=[
            jax.ShapeDtypeStruct((NSLOT, NQ), jnp.int32),
            jax.ShapeDtypeStruct((NSLOT, NQ), jnp.float32),
            jax.ShapeDtypeStruct((1, NQ), jnp.float32),
        ],
    )(qT)


def _sc_gather(table, idx):
    B = idx.shape[0]
    NW = 32
    BPW = B // NW
    CH = 128
    mesh = plsc.VectorSubcoreMesh(core_axis_name="c", subcore_axis_name="s")

    NCH = BPW // CH
    idx3 = idx.reshape(NW, NCH, CH)

    @functools.partial(
        pl.kernel, mesh=mesh,
        out_type=jax.ShapeDtypeStruct((B, C), jnp.float32),
        scratch_types=[
            pltpu.VMEM((NCH, CH), jnp.int32),
            pltpu.VMEM((CH, C), jnp.float32),
            pltpu.VMEM((CH, C), jnp.float32),
            pltpu.SemaphoreType.DMA,
            pltpu.SemaphoreType.DMA,
        ],
    )
    def k(table_hbm, idx_hbm, out_hbm, idx_v, rows0, rows1, g0, g1):
        wid = lax.axis_index("s") * 2 + lax.axis_index("c")
        base = wid * BPW
        pltpu.sync_copy(idx_hbm.at[wid], idx_v)
        bufs = ((rows0, g0), (rows1, g1))

        def gather_start(t, rows, sem):
            pltpu.make_async_copy(
                table_hbm.at[idx_v.at[t]], rows, sem).start()

        for b in range(2):
            gather_start(b, *bufs[b])

        @pl.loop(0, NCH, step=2)
        def _(t):
            for b in range(2):
                rows, sem = bufs[b]
                tt = t + b
                pltpu.make_async_copy(
                    table_hbm.at[idx_v.at[tt]], rows, sem).wait()
                pltpu.sync_copy(rows, out_hbm.at[pl.ds(base + tt * CH, CH)])

                @pl.when(tt + 2 < NCH)
                def _():
                    gather_start(tt + 2, rows, sem)

    return k(table, idx3)


def _gelu(x):
    return x * 0.5 * (1.0 + lax.erf(x * _SQRT1_2))


def _mlp_kernel(q_ref, nidxT_ref, smaskT_ref, scnt_ref, fN_ref, W1_ref, b1_ref,
                W2_ref, b2_ref, W3_ref, b3_ref, Wp_ref, bp_ref, out_ref):
    q = q_ref[...]
    scnt = scnt_ref[...]
    W1 = W1_ref[...]
    W2 = W2_ref[...]
    W3 = W3_ref[...]
    xp = (q[:, 0:1] * W1[3:4, :] + q[:, 1:2] * W1[4:5, :]
          + q[:, 2:3] * W1[5:6, :] + b1_ref[...])
    acc = jnp.zeros((QB, C), jnp.float32)
    for s in range(NSLOT):
        fl = nidxT_ref[:, s:s + 1]
        ixf = (fl >> 10).astype(jnp.float32)
        iyf = ((fl >> 5) & 31).astype(jnp.float32)
        izf = (fl & 31).astype(jnp.float32)
        h = (xp + (ixf * STEP) * W1[0:1, :] + (iyf * STEP) * W1[1:2, :]
             + (izf * STEP) * W1[2:3, :])
        h = _gelu(h)
        h = _gelu(jnp.dot(h, W2, preferred_element_type=jnp.float32)
                  + b2_ref[...])
        kern = (jnp.dot(h, W3, preferred_element_type=jnp.float32)
                + b3_ref[...])
        m = smaskT_ref[:, s:s + 1]
        acc = acc + kern * fN_ref[s] * m
    agg = acc / jnp.maximum(scnt, 1.0)
    out_ref[...] = (jnp.dot(agg, Wp_ref[...], preferred_element_type=jnp.float32)
                    + bp_ref[...])


def _mlp(q, nidxT, smaskT, scntT, fN, W1, b1, W2, b2, W3, b3, Wp, bp):
    grid = (NQ // QB,)
    return pl.pallas_call(
        _mlp_kernel,
        grid=grid,
        in_specs=[
            pl.BlockSpec((QB, 3), lambda i: (i, 0)),
            pl.BlockSpec((QB, NSLOT), lambda i: (i, 0)),
            pl.BlockSpec((QB, NSLOT), lambda i: (i, 0)),
            pl.BlockSpec((QB, 1), lambda i: (i, 0)),
            pl.BlockSpec((NSLOT, QB, C), lambda i: (0, i, 0)),
            pl.BlockSpec((6, H1), lambda i: (0, 0)),
            pl.BlockSpec((1, H1), lambda i: (0, 0)),
            pl.BlockSpec((H1, C), lambda i: (0, 0)),
            pl.BlockSpec((1, C), lambda i: (0, 0)),
            pl.BlockSpec((C, C), lambda i: (0, 0)),
            pl.BlockSpec((1, C), lambda i: (0, 0)),
            pl.BlockSpec((C, 4), lambda i: (0, 0)),
            pl.BlockSpec((1, 4), lambda i: (0, 0)),
        ],
        out_specs=pl.BlockSpec((QB, 4), lambda i: (i, 0)),
        out_shape=jax.ShapeDtypeStruct((NQ, 4), jnp.float32),
    )(q, nidxT, smaskT, scntT, fN, W1, b1, W2, b2, W3, b3, Wp, bp)


def kernel(latent_embed, latent_queries, output_queries,
           W1, b1, W2, b2, W3, b3, Wp, bp):
    del latent_queries
    q = output_queries[0]
    f_y = latent_embed.reshape(-1, C)
    nidx, smask, scnt = _search(q.T)
    fN = _sc_gather(f_y, nidx.reshape(-1))
    out = _mlp(q, nidx.T, smask.T, scnt.reshape(NQ, 1),
               fN.reshape(NSLOT, NQ, C), W1, b1.reshape(1, H1), W2,
               b2.reshape(1, C), W3, b3.reshape(1, C), Wp, bp.reshape(1, 4))
    return out[None]
